# baseline (speedup 1.0000x reference)
import functools

import jax
import jax.numpy as jnp
from jax import lax
from jax.experimental import pallas as pl
from jax.experimental.pallas import tpu as pltpu
from jax.experimental.pallas import tpu_sc as plsc

_VOCAB = 1000
_D = 1000
_B = 51200
_NC, _NS = 2, 16
_NW = _NC * _NS
_BPW = _B // _NW
_CH = 32
_NCHUNK = _BPW // _CH


@functools.partial(
    pl.kernel,
    mesh=plsc.VectorSubcoreMesh(core_axis_name="c", subcore_axis_name="s"),
    out_type=jax.ShapeDtypeStruct((_B, _D), jnp.float32),
    scratch_types=[
        pltpu.VMEM((_BPW,), jnp.int32),
        pltpu.VMEM((_CH, _D), jnp.float32),
        pltpu.SemaphoreType.DMA,
    ],
    compiler_params=pltpu.CompilerParams(use_tc_tiling_on_sc=False),
)
def _gather(table_hbm, idx_hbm, out_hbm, idx_v, rows_v, sem):
    wid = lax.axis_index("s") * _NC + lax.axis_index("c")
    base = wid * _BPW
    pltpu.sync_copy(idx_hbm.at[pl.ds(base, _BPW)], idx_v)

    def step(i, carry):
        off = i * _CH
        pltpu.async_copy(
            table_hbm.at[idx_v.at[pl.ds(off, _CH)]], rows_v, sem
        ).wait()
        pltpu.sync_copy(rows_v, out_hbm.at[pl.ds(base + off, _CH)])
        return carry

    lax.fori_loop(0, _NCHUNK, step, 0)


def kernel(idx, table):
    idx_flat = idx.reshape(-1).astype(jnp.int32)
    out = _gather(table, idx_flat)
    return out.reshape(idx.shape + (_VOCAB,))


# speedup vs baseline: 1.0542x; 1.0542x over previous
import functools

import jax
import jax.numpy as jnp
from jax import lax
from jax.experimental import pallas as pl
from jax.experimental.pallas import tpu as pltpu
from jax.experimental.pallas import tpu_sc as plsc

_VOCAB = 1000
_D = 1000
_B = 51200
_NC, _NS = 2, 16
_NW = _NC * _NS
_BPW = _B // _NW
_CH = 16
_NCHUNK = _BPW // _CH
_NBUF = 4
assert _NCHUNK % _NBUF == 0


@functools.partial(
    pl.kernel,
    mesh=plsc.VectorSubcoreMesh(core_axis_name="c", subcore_axis_name="s"),
    out_type=jax.ShapeDtypeStruct((_B, _D), jnp.float32),
    scratch_types=[
        pltpu.VMEM((_BPW,), jnp.int32),
        pltpu.VMEM((_NBUF, _CH, _D), jnp.float32),
        pltpu.SemaphoreType.DMA((_NBUF,)),
        pltpu.SemaphoreType.DMA((_NBUF,)),
    ],
    compiler_params=pltpu.CompilerParams(use_tc_tiling_on_sc=False),
)
def _gather(table_hbm, idx_hbm, out_hbm, idx_v, rows_v, gsem, osem):
    wid = lax.axis_index("s") * _NC + lax.axis_index("c")
    base = wid * _BPW
    pltpu.sync_copy(idx_hbm.at[pl.ds(base, _BPW)], idx_v)

    def fire_gather(i, b):
        pltpu.make_async_copy(
            table_hbm.at[idx_v.at[pl.ds(i * _CH, _CH)]],
            rows_v.at[b],
            gsem.at[b],
        ).start()

    def wait_gather(i, b):
        pltpu.make_async_copy(
            table_hbm.at[idx_v.at[pl.ds(i * _CH, _CH)]],
            rows_v.at[b],
            gsem.at[b],
        ).wait()

    def out_copy(i, b):
        return pltpu.make_async_copy(
            rows_v.at[b],
            out_hbm.at[pl.ds(base + i * _CH, _CH)],
            osem.at[b],
        )

    for b in range(_NBUF):
        fire_gather(b, b)

    def outer(g, carry):
        i0 = g * _NBUF
        for b in range(_NBUF):
            i = i0 + b
            wait_gather(i, b)
            out_copy(i, b).start()

            @pl.when(i + _NBUF < _NCHUNK)
            def _():
                out_copy(i, b).wait()
                fire_gather(i + _NBUF, b)

        return carry

    lax.fori_loop(0, _NCHUNK // _NBUF, outer, 0)

    for b in range(_NBUF):
        out_copy(_NCHUNK - _NBUF + b, b).wait()


def kernel(idx, table):
    idx_flat = idx.reshape(-1).astype(jnp.int32)
    out = _gather(table, idx_flat)
    return out.reshape(idx.shape + (_VOCAB,))


# speedup vs baseline: 1.0548x; 1.0005x over previous
import functools

import jax
import jax.numpy as jnp
from jax import lax
from jax.experimental import pallas as pl
from jax.experimental.pallas import tpu as pltpu
from jax.experimental.pallas import tpu_sc as plsc

_VOCAB = 1000
_D = 1000
_BATCH = 1024
_SEQ = 50
_NC, _NS = 2, 16
_NW = _NC * _NS
_BPW = _BATCH // _NW
_NBUF = 2
assert _BPW % _NBUF == 0


@functools.partial(
    pl.kernel,
    mesh=plsc.VectorSubcoreMesh(core_axis_name="c", subcore_axis_name="s"),
    out_type=jax.ShapeDtypeStruct((_BATCH, _SEQ, _D), jnp.float32),
    scratch_types=[
        pltpu.VMEM((_BPW, _SEQ), jnp.int32),
        pltpu.VMEM((_NBUF, _SEQ, _D), jnp.float32),
        pltpu.SemaphoreType.DMA((_NBUF,)),
        pltpu.SemaphoreType.DMA((_NBUF,)),
    ],
    compiler_params=pltpu.CompilerParams(use_tc_tiling_on_sc=False),
)
def _gather(table_hbm, idx_hbm, out_hbm, idx_v, rows_v, gsem, osem):
    wid = lax.axis_index("s") * _NC + lax.axis_index("c")
    base = wid * _BPW
    pltpu.sync_copy(idx_hbm.at[pl.ds(base, _BPW)], idx_v)

    def gather_copy(j, b):
        return pltpu.make_async_copy(
            table_hbm.at[idx_v.at[j]],
            rows_v.at[b],
            gsem.at[b],
        )

    def out_copy(j, b):
        return pltpu.make_async_copy(
            rows_v.at[b],
            out_hbm.at[base + j],
            osem.at[b],
        )

    for b in range(_NBUF):
        gather_copy(b, b).start()

    def outer(g, carry):
        j0 = g * _NBUF
        for b in range(_NBUF):
            j = j0 + b
            gather_copy(j, b).wait()
            out_copy(j, b).start()

            @pl.when(j + _NBUF < _BPW)
            def _():
                out_copy(j, b).wait()
                gather_copy(j + _NBUF, b).start()

        return carry

    lax.fori_loop(0, _BPW // _NBUF, outer, 0)

    for b in range(_NBUF):
        out_copy(_BPW - _NBUF + b, b).wait()


def kernel(idx, table):
    return _gather(table, idx.astype(jnp.int32))


# speedup vs baseline: 1.2037x; 1.1412x over previous
import functools

import jax
import jax.numpy as jnp
from jax import lax
from jax.experimental import pallas as pl
from jax.experimental.pallas import tpu as pltpu
from jax.experimental.pallas import tpu_sc as plsc

_VOCAB = 1000
_D = 1000
_BATCH = 1024
_SEQ = 50
_NTV = 125
_NC, _NS = 2, 16
_NW = _NC * _NS
_TPW = 4
_NGRP = _BATCH // 16


@functools.partial(
    pl.kernel,
    mesh=plsc.VectorSubcoreMesh(core_axis_name="c", subcore_axis_name="s"),
    out_type=jax.ShapeDtypeStruct((_SEQ, _NTV, 8, 8, 128), jnp.float32),
    scratch_types=[
        pltpu.VMEM((_TPW, 8 * _VOCAB), jnp.float32),
        pltpu.VMEM((2, _TPW, 8, 8, 128), jnp.float32),
        pltpu.VMEM((_BATCH,), jnp.int32),
        pltpu.SemaphoreType.DMA((2, _TPW)),
    ],
    compiler_params=pltpu.CompilerParams(
        use_tc_tiling_on_sc=False, needs_layout_passes=False
    ),
)
def _gather(table_hbm, idx_hbm, out_hbm, tcols, obuf, ibuf, osem):
    wid = lax.axis_index("s") * _NC + lax.axis_index("c")
    tv0 = wid * _TPW
    pltpu.sync_copy(table_hbm.at[pl.ds(tv0, _TPW)], tcols)

    def out_dma(s, db, t):
        return pltpu.make_async_copy(
            obuf.at[db, t],
            out_hbm.at[s, tv0 + t],
            osem.at[db, t],
        )

    def do_s(s, db):
        pltpu.sync_copy(idx_hbm.at[s], ibuf)

        @pl.when(s >= 2)
        def _():
            for t in range(_TPW):
                @pl.when(tv0 + t < _NTV)
                def _():
                    out_dma(s - 2, db, t).wait()

        def g_body(g, carry):
            tb = g // 8
            lane0 = 16 * (g % 8)
            iv = ibuf[pl.ds(16 * g, 16)]
            for sub in range(8):
                fidx = iv + jnp.int32(_VOCAB * sub)
                for t in range(_TPW):
                    v = plsc.load_gather(tcols.at[t], [fidx])
                    obuf[db, t, tb, sub, pl.ds(lane0, 16)] = v
            return carry

        lax.fori_loop(0, _NGRP, g_body, 0)

        for t in range(_TPW):
            @pl.when(tv0 + t < _NTV)
            def _():
                out_dma(s, db, t).start()

    def outer(s2, carry):
        for db in range(2):
            do_s(s2 * 2 + db, db)
        return carry

    lax.fori_loop(0, _SEQ // 2, outer, 0)

    for db in range(2):
        for t in range(_TPW):
            @pl.when(tv0 + t < _NTV)
            def _():
                out_dma(_SEQ - 2 + db, db, t).wait()


def kernel(idx, table):
    idx_t = jnp.transpose(idx).astype(jnp.int32)
    table_t = jnp.transpose(table).reshape(_NTV, 8 * _VOCAB)
    table_p = jnp.pad(table_t, ((0, _NW * _TPW - _NTV), (0, 0)))
    out5d = _gather(table_p, idx_t)
    t = jnp.transpose(out5d, (2, 4, 0, 1, 3))
    return t.reshape(_BATCH, _SEQ, _D)


# speedup vs baseline: 1.4503x; 1.2049x over previous
import functools

import jax
import jax.numpy as jnp
from jax import lax
from jax.experimental import pallas as pl
from jax.experimental.pallas import tpu as pltpu
from jax.experimental.pallas import tpu_sc as plsc

_VOCAB = 1000
_D = 1000
_BATCH = 1024
_SEQ = 50
_NTV = 125
_NC, _NS = 2, 16
_NW = _NC * _NS
_TPW = 4


@functools.partial(
    pl.kernel,
    mesh=plsc.VectorSubcoreMesh(core_axis_name="c", subcore_axis_name="s"),
    out_type=jax.ShapeDtypeStruct((_SEQ, _NTV, 8, 8, 128), jnp.float32),
    scratch_types=[
        pltpu.VMEM((_TPW, 8 * _VOCAB), jnp.float32),
        pltpu.VMEM((2, _TPW, 8, 8, 128), jnp.float32),
        pltpu.VMEM((2, _BATCH), jnp.int32),
        pltpu.SemaphoreType.DMA((2, _TPW)),
        pltpu.SemaphoreType.DMA((2,)),
    ],
    compiler_params=pltpu.CompilerParams(
        use_tc_tiling_on_sc=False, needs_layout_passes=False
    ),
)
def _gather(table_hbm, idx_hbm, out_hbm, tcols, obuf, ibuf, osem, isem):
    wid = lax.axis_index("s") * _NC + lax.axis_index("c")
    tv0 = wid * _TPW
    pltpu.sync_copy(table_hbm.at[pl.ds(tv0, _TPW)], tcols)

    def idx_dma(s, db):
        return pltpu.make_async_copy(
            idx_hbm.at[s], ibuf.at[db], isem.at[db]
        )

    def out_dma(s, db, t):
        return pltpu.make_async_copy(
            obuf.at[db, t],
            out_hbm.at[s, tv0 + t],
            osem.at[db, t],
        )

    idx_dma(0, 0).start()

    def do_s(s, db):
        idx_dma(s, db).wait()

        @pl.when(s + 1 < _SEQ)
        def _():
            idx_dma(s + 1, 1 - db).start()

        @pl.when(s >= 2)
        def _():
            for t in range(_TPW):
                @pl.when(tv0 + t < _NTV)
                def _():
                    out_dma(s - 2, db, t).wait()

        @plsc.parallel_loop(0, 8, unroll=2)
        def _tb_body(tb):
            for gg in range(8):
                iv = ibuf[db, pl.ds(128 * tb + 16 * gg, 16)]
                for sub in range(8):
                    fidx = iv + jnp.int32(_VOCAB * sub)
                    for t in range(_TPW):
                        v = plsc.load_gather(tcols.at[t], [fidx])
                        obuf[db, t, tb, sub, pl.ds(16 * gg, 16)] = v

        for t in range(_TPW):
            @pl.when(tv0 + t < _NTV)
            def _():
                out_dma(s, db, t).start()

    def outer(s2, carry):
        for db in range(2):
            do_s(s2 * 2 + db, db)
        return carry

    lax.fori_loop(0, _SEQ // 2, outer, 0)

    for db in range(2):
        for t in range(_TPW):
            @pl.when(tv0 + t < _NTV)
            def _():
                out_dma(_SEQ - 2 + db, db, t).wait()


def kernel(idx, table):
    idx_t = jnp.transpose(idx).astype(jnp.int32)
    table_t = jnp.transpose(table).reshape(_NTV, 8 * _VOCAB)
    table_p = jnp.pad(table_t, ((0, _NW * _TPW - _NTV), (0, 0)))
    out5d = _gather(table_p, idx_t)
    t = jnp.transpose(out5d, (2, 4, 0, 1, 3))
    return t.reshape(_BATCH, _SEQ, _D)


# speedup vs baseline: 3.1337x; 2.1607x over previous
import functools

import jax
import jax.numpy as jnp
from jax import lax
from jax.experimental import pallas as pl
from jax.experimental.pallas import tpu as pltpu
from jax.experimental.pallas import tpu_sc as plsc

_VOCAB = 1000
_D = 1000
_BATCH = 1024
_SEQ = 50
_NTV = 125
_NC, _NS = 2, 16
_NW = _NC * _NS
_TPW = 4


@functools.partial(
    pl.kernel,
    mesh=plsc.VectorSubcoreMesh(core_axis_name="c", subcore_axis_name="s"),
    out_type=jax.ShapeDtypeStruct((_SEQ, _NTV, 8, 8, 128), jnp.float32),
    scratch_types=[
        pltpu.VMEM((_TPW, 8 * _VOCAB), jnp.float32),
        pltpu.VMEM((2, _TPW, 8, 8, 128), jnp.float32),
        pltpu.VMEM((2, _BATCH), jnp.int32),
        pltpu.SemaphoreType.DMA((2, _TPW)),
        pltpu.SemaphoreType.DMA((2,)),
    ],
    compiler_params=pltpu.CompilerParams(
        use_tc_tiling_on_sc=False, needs_layout_passes=False
    ),
)
def _gather(table_hbm, idx_hbm, out_hbm, tcols, obuf, ibuf, osem, isem):
    wid = lax.axis_index("s") * _NC + lax.axis_index("c")
    tv0 = wid * _TPW
    pltpu.sync_copy(table_hbm.at[pl.ds(tv0, _TPW)], tcols)

    def idx_dma(s, db):
        return pltpu.make_async_copy(
            idx_hbm.at[s], ibuf.at[db], isem.at[db]
        )

    def out_dma(s, db, t):
        return pltpu.make_async_copy(
            obuf.at[db, t],
            out_hbm.at[s, tv0 + t],
            osem.at[db, t],
        )

    idx_dma(0, 0).start()

    def do_s(s, db):
        idx_dma(s, db).wait()

        @pl.when(s + 1 < _SEQ)
        def _():
            idx_dma(s + 1, 1 - db).start()

        @pl.when(s >= 2)
        def _():
            for t in range(_TPW):
                @pl.when(tv0 + t < _NTV)
                def _():
                    out_dma(s - 2, db, t).wait()

        @plsc.parallel_loop(0, 8, unroll=2)
        def _tb_body(tb):
            for gg in range(8):
                iv = ibuf[db, pl.ds(128 * tb + 16 * gg, 16)]
                vals = []
                for sub in range(8):
                    fidx = iv + jnp.int32(_VOCAB * sub)
                    for t in range(_TPW):
                        vals.append(
                            (sub, t, plsc.load_gather(tcols.at[t], [fidx]))
                        )
                for sub, t, v in vals:
                    obuf[db, t, tb, sub, pl.ds(16 * gg, 16)] = v

        for t in range(_TPW):
            @pl.when(tv0 + t < _NTV)
            def _():
                out_dma(s, db, t).start()

    def outer(s2, carry):
        for db in range(2):
            do_s(s2 * 2 + db, db)
        return carry

    lax.fori_loop(0, _SEQ // 2, outer, 0)

    for db in range(2):
        for t in range(_TPW):
            @pl.when(tv0 + t < _NTV)
            def _():
                out_dma(_SEQ - 2 + db, db, t).wait()


def kernel(idx, table):
    idx_t = jnp.transpose(idx).astype(jnp.int32)
    table_t = jnp.transpose(table).reshape(_NTV, 8 * _VOCAB)
    table_p = jnp.pad(table_t, ((0, _NW * _TPW - _NTV), (0, 0)))
    out5d = _gather(table_p, idx_t)
    t = jnp.transpose(out5d, (2, 4, 0, 1, 3))
    return t.reshape(_BATCH, _SEQ, _D)


# speedup vs baseline: 4.2892x; 1.3687x over previous
import functools

import jax
import jax.numpy as jnp
from jax import lax
from jax.experimental import pallas as pl
from jax.experimental.pallas import tpu as pltpu
from jax.experimental.pallas import tpu_sc as plsc

_VOCAB = 1000
_D = 1000
_BATCH = 1024
_SEQ = 50
_NTV = 125
_NC, _NS = 2, 16
_NW = _NC * _NS
_TPW = 4


@functools.partial(
    pl.kernel,
    mesh=plsc.VectorSubcoreMesh(core_axis_name="c", subcore_axis_name="s"),
    out_type=jax.ShapeDtypeStruct((_SEQ, _NTV, 8, 8, 128), jnp.float32),
    scratch_types=[
        pltpu.VMEM((_TPW, 8 * _VOCAB), jnp.float32),
        pltpu.VMEM((2, _TPW, 8, 8, 128), jnp.float32),
        pltpu.VMEM((2, _BATCH), jnp.int32),
        pltpu.SemaphoreType.DMA((2, _TPW)),
        pltpu.SemaphoreType.DMA((2,)),
    ],
    compiler_params=pltpu.CompilerParams(
        use_tc_tiling_on_sc=False, needs_layout_passes=False
    ),
)
def _gather(table_hbm, idx_hbm, out_hbm, tcols, obuf, ibuf, osem, isem):
    wid = lax.axis_index("s") * _NC + lax.axis_index("c")
    tv0 = wid * _TPW
    pltpu.sync_copy(table_hbm.at[pl.ds(tv0, _TPW)], tcols)

    def idx_dma(s, db):
        return pltpu.make_async_copy(
            idx_hbm.at[s], ibuf.at[db], isem.at[db]
        )

    def out_dma(s, db, t):
        return pltpu.make_async_copy(
            obuf.at[db, t],
            out_hbm.at[s, tv0 + t],
            osem.at[db, t],
        )

    idx_dma(0, 0).start()

    def do_s(s, db):
        idx_dma(s, db).wait()

        @pl.when(s + 1 < _SEQ)
        def _():
            idx_dma(s + 1, 1 - db).start()

        @pl.when(s >= 2)
        def _():
            for t in range(_TPW):
                @pl.when(tv0 + t < _NTV)
                def _():
                    out_dma(s - 2, db, t).wait()

        @plsc.parallel_loop(0, 64, unroll=2)
        def _g_body(g):
            tb = g // 8
            gg = g % 8
            iv = ibuf[db, pl.ds(16 * g, 16)]
            vals = []
            for sub in range(8):
                fidx = iv + jnp.int32(_VOCAB * sub)
                for t in range(_TPW):
                    vals.append(
                        (sub, t, plsc.load_gather(tcols.at[t], [fidx]))
                    )
            for sub, t, v in vals:
                obuf[db, t, tb, sub, pl.ds(16 * gg, 16)] = v

        for t in range(_TPW):
            @pl.when(tv0 + t < _NTV)
            def _():
                out_dma(s, db, t).start()

    def outer(s2, carry):
        for db in range(2):
            do_s(s2 * 2 + db, db)
        return carry

    lax.fori_loop(0, _SEQ // 2, outer, 0)

    for db in range(2):
        for t in range(_TPW):
            @pl.when(tv0 + t < _NTV)
            def _():
                out_dma(_SEQ - 2 + db, db, t).wait()


def kernel(idx, table):
    idx_t = jnp.transpose(idx).astype(jnp.int32)
    table_t = jnp.transpose(table).reshape(_NTV, 8 * _VOCAB)
    table_p = jnp.pad(table_t, ((0, _NW * _TPW - _NTV), (0, 0)))
    out5d = _gather(table_p, idx_t)
    t = jnp.transpose(out5d, (2, 4, 0, 1, 3))
    return t.reshape(_BATCH, _SEQ, _D)
